# baseline (speedup 1.0000x reference)
import functools

import jax
import jax.numpy as jnp
from jax import lax
from jax.experimental import pallas as pl
from jax.experimental.pallas import tpu as pltpu
from jax.experimental.pallas import tpu_sc as plsc

_NC = 2
_NS = 16
_CHUNK = 32
_NBUF = 4


def _ring_body(idx_v, gathers, writes, per_w, n_grp):
    n_ch = n_grp * _NBUF
    for j in range(_NBUF - 1):
        for gth in gathers(j, j):
            gth.start()

    def body(g, _):
        for b in range(_NBUF):
            j = g * _NBUF + b
            for gth in gathers(j, b):
                gth.wait()
            for w in writes(j, b):
                w.start()
            bp = (b - 1) % _NBUF

            def drain_prev():
                for w in writes(j - 1, bp):
                    w.wait()

            def refill():
                drain_prev()
                for gth in gathers(j + _NBUF - 1, bp):
                    gth.start()

            if b == 0:
                pl.when(g >= 1)(drain_prev)
                for gth in gathers(j + _NBUF - 1, bp):
                    gth.start()
            else:
                pl.when(g < n_grp - 1)(refill)
        return ()

    lax.fori_loop(0, n_grp, body, (), unroll=False)
    for j in range(n_ch - _NBUF, n_ch):
        for w in writes(j, j % _NBUF):
            w.wait()


@functools.lru_cache(maxsize=None)
def _make_col_gather(M, D, n_loc, width, tc_tiling):
    NW = _NC * _NS
    per_w = n_loc // NW
    n_ch = per_w // _CHUNK
    assert n_ch % _NBUF == 0 and n_ch >= 2 * _NBUF
    n_grp = n_ch // _NBUF
    mesh = plsc.VectorSubcoreMesh(core_axis_name="c", subcore_axis_name="s")

    @functools.partial(
        pl.kernel,
        mesh=mesh,
        out_type=jax.ShapeDtypeStruct((n_loc, width), jnp.float32),
        scratch_types=[
            pltpu.VMEM((per_w,), jnp.int32),
            [pltpu.VMEM((_CHUNK, width), jnp.float32) for _ in range(_NBUF)],
            [pltpu.SemaphoreType.DMA for _ in range(_NBUF)],
            [pltpu.SemaphoreType.DMA for _ in range(_NBUF)],
        ],
        compiler_params=pltpu.CompilerParams(use_tc_tiling_on_sc=tc_tiling),
    )
    def gather_kernel(src_hbm, loc_hbm, out_hbm, idx_v, bufs, gsems, wsems):
        wid = lax.axis_index("s") * _NC + lax.axis_index("c")
        base = wid * per_w

        def gathers(j, b):
            idx = idx_v.at[pl.ds(j * _CHUNK, _CHUNK)]
            if width == D:
                src = src_hbm.at[idx]
            else:
                src = src_hbm.at[idx, pl.ds(0, width)]
            return (pltpu.make_async_copy(src, bufs[b], gsems[b]),)

        def writes(j, b):
            row0 = base + j * _CHUNK
            return (
                pltpu.make_async_copy(
                    bufs[b],
                    out_hbm.at[pl.ds(row0, _CHUNK), pl.ds(0, width)],
                    wsems[b]),
            )

        pltpu.sync_copy(loc_hbm.at[pl.ds(base, per_w)], idx_v)
        _ring_body(idx_v, gathers, writes, per_w, n_grp)

    return gather_kernel


def kernel(kv_buffer, loc, cache_k_nope, cache_k_rope):
    M, D = kv_buffer.shape
    n_loc = loc.shape[0]
    nope_dim = cache_k_nope.shape[-1]
    rope_dim = cache_k_rope.shape[-1]
    rope_src = lax.slice(kv_buffer, (0, nope_dim), (M, nope_dim + rope_dim))
    nope = _make_col_gather(M, D, n_loc, nope_dim, True)(kv_buffer, loc)
    rope = _make_col_gather(M, rope_dim, n_loc, rope_dim, False)(rope_src, loc)
    return nope, rope
